# baseline (speedup 1.0000x reference)
import functools

import jax
import jax.numpy as jnp
import numpy as np
from jax import lax
from jax.experimental import pallas as pl
from jax.experimental.pallas import tpu as pltpu
from jax.experimental.pallas import tpu_sc as plsc

K = 3
DIM = 3
KD = K ** DIM
NC = 2
NS = 16
NW = NC * NS


def _basis(attr_r, src_r, E):
    _, R, EC = attr_r.shape

    def body(a_ref, s_ref, b_ref, rid_ref):
        src = s_ref[...]
        fracs = []
        los = []
        for d in range(DIM):
            v = a_ref[d] * float(K)
            lo = jnp.floor(v)
            fracs.append(v - lo)
            los.append(lo.astype(jnp.int32))
        e_id = (lax.broadcasted_iota(jnp.int32, (R, EC), 0) * EC
                + lax.broadcasted_iota(jnp.int32, (R, EC), 1))
        valid = e_id < E
        for s in range(8):
            b = None
            idx = None
            for d in range(DIM):
                bit = (s >> d) & 1
                f = fracs[d] if bit else 1.0 - fracs[d]
                b = f if b is None else b * f
                t = lax.rem(los[d] + bit, K) * (K ** d)
                idx = t if idx is None else idx + t
            b = jnp.where(valid, b, 0.0)
            b_ref[s] = b
            rid_ref[s] = src * KD + idx

    return pl.pallas_call(
        body,
        out_shape=[jax.ShapeDtypeStruct((8, R, EC), jnp.float32),
                   jax.ShapeDtypeStruct((8, R, EC), jnp.int32)],
    )(attr_r, src_r)


def _mm(x, Wfs, root, bias, block_n):
    N, Cin = x.shape
    KOs = [Wf.shape[1] for Wf in Wfs]
    C = root.shape[1]
    nw = len(Wfs)

    def body(x_ref, *refs):
        w_refs = refs[:nw]
        rt_ref, b_ref = refs[nw], refs[nw + 1]
        y_refs = refs[nw + 2:nw + 2 + nw]
        r_ref = refs[-1]
        xb = x_ref[...]
        for w_ref, y_ref in zip(w_refs, y_refs):
            y_ref[...] = jnp.dot(
                xb, w_ref[...],
                preferred_element_type=jnp.float32).astype(jnp.bfloat16)
        r_ref[...] = (jnp.dot(xb, rt_ref[...], preferred_element_type=jnp.float32)
                      + b_ref[...])

    return pl.pallas_call(
        body,
        grid=(N // block_n,),
        in_specs=[pl.BlockSpec((block_n, Cin), lambda i: (i, 0))]
        + [pl.BlockSpec((Cin, KO), lambda i: (0, 0)) for KO in KOs]
        + [
            pl.BlockSpec((Cin, C), lambda i: (0, 0)),
            pl.BlockSpec((1, C), lambda i: (0, 0)),
        ],
        out_specs=[pl.BlockSpec((block_n, KO), lambda i: (i, 0)) for KO in KOs]
        + [pl.BlockSpec((block_n, C), lambda i: (i, 0))],
        out_shape=[jax.ShapeDtypeStruct((N, KO), jnp.bfloat16) for KO in KOs]
        + [jax.ShapeDtypeStruct((N, C), jnp.float32)],
    )(x, *Wfs, root, bias)


def _fuse1(parts, r, g, be, W2f, root2, b2, C, BN):
    N = r.shape[0]
    _, N_acc, C_msg = parts.shape
    KO = W2f.shape[1]
    C2 = root2.shape[1]

    def body(p_ref, r_ref, g_ref, be_ref, w_ref, rt_ref, b2_ref,
             y_ref, r2_ref, ci_ref, h_sc):
        i = pl.program_id(0)

        @pl.when(i == 0)
        def _():
            p0 = p_ref[0, :N].astype(jnp.float32)
            p1 = p_ref[1, :N].astype(jnp.float32)
            cnt = p0[:, C:C + 1] + p1[:, C:C + 1]
            cinv = 1.0 / jnp.maximum(cnt, 1.0)
            out = (p0[:, :C] + p1[:, :C]) * cinv + r_ref[...]
            m = jnp.mean(out, axis=0, keepdims=True)
            var = jnp.mean(out * out, axis=0, keepdims=True) - m * m
            xn = (out - m) * lax.rsqrt(var + 1e-5) * g_ref[...] + be_ref[...]
            h = jnp.where(xn > 0, xn, jnp.exp(xn) - 1.0)
            h_sc[...] = h
            ci_ref[...] = cinv
            r2_ref[...] = (jnp.dot(h, rt_ref[...],
                                   preferred_element_type=jnp.float32)
                           + b2_ref[...])
        y_ref[...] = jnp.dot(
            h_sc[pl.ds(i * BN, BN), :], w_ref[...],
            preferred_element_type=jnp.float32).astype(jnp.bfloat16)

    return pl.pallas_call(
        body,
        grid=(N // BN,),
        in_specs=[
            pl.BlockSpec((2, N_acc, C_msg), lambda i: (0, 0, 0)),
            pl.BlockSpec((N, C), lambda i: (0, 0)),
            pl.BlockSpec((1, C), lambda i: (0, 0)),
            pl.BlockSpec((1, C), lambda i: (0, 0)),
            pl.BlockSpec((C, KO), lambda i: (0, 0)),
            pl.BlockSpec((C, C2), lambda i: (0, 0)),
            pl.BlockSpec((1, C2), lambda i: (0, 0)),
        ],
        out_specs=[
            pl.BlockSpec((BN, KO), lambda i: (i, 0)),
            pl.BlockSpec((N, C2), lambda i: (0, 0)),
            pl.BlockSpec((N, 1), lambda i: (0, 0)),
        ],
        out_shape=[jax.ShapeDtypeStruct((N, KO), jnp.bfloat16),
                   jax.ShapeDtypeStruct((N, C2), jnp.float32),
                   jax.ShapeDtypeStruct((N, 1), jnp.float32)],
        scratch_shapes=[pltpu.VMEM((N, C), jnp.float32)],
    )(parts, r, g, be, W2f, root2, b2)


def _fuse2(parts, r, cinv, g, be, x, W3fa, W3fb, root3, b3, C, BN):
    N = r.shape[0]
    _, N_acc, C_msg = parts.shape
    KO = W3fa.shape[1]
    C3 = root3.shape[1]
    Cc = C + x.shape[1]

    def body(p_ref, r_ref, ci_ref, g_ref, be_ref, x_ref, wa_ref, wb_ref,
             rt_ref, b3_ref, ya_ref, yb_ref, r3_ref, h_sc):
        i = pl.program_id(0)

        @pl.when(i == 0)
        def _():
            out = ((p_ref[0, :N].astype(jnp.float32)
                    + p_ref[1, :N].astype(jnp.float32)) * ci_ref[...]
                   + r_ref[...])
            m = jnp.mean(out, axis=0, keepdims=True)
            var = jnp.mean(out * out, axis=0, keepdims=True) - m * m
            xn = (out - m) * lax.rsqrt(var + 1e-5) * g_ref[...] + be_ref[...]
            h2 = jnp.where(xn > 0, xn, jnp.exp(xn) - 1.0)
            hc = jnp.concatenate([h2, x_ref[...]], axis=1)
            h_sc[...] = hc
            r3_ref[...] = (jnp.dot(hc, rt_ref[...],
                                   preferred_element_type=jnp.float32)
                           + b3_ref[...])
        hc = h_sc[pl.ds(i * BN, BN), :]
        ya_ref[...] = jnp.dot(
            hc, wa_ref[...],
            preferred_element_type=jnp.float32).astype(jnp.bfloat16)
        yb_ref[...] = jnp.dot(
            hc, wb_ref[...],
            preferred_element_type=jnp.float32).astype(jnp.bfloat16)

    return pl.pallas_call(
        body,
        grid=(N // BN,),
        in_specs=[
            pl.BlockSpec((2, N_acc, C_msg), lambda i: (0, 0, 0)),
            pl.BlockSpec((N, C), lambda i: (0, 0)),
            pl.BlockSpec((N, 1), lambda i: (0, 0)),
            pl.BlockSpec((1, C), lambda i: (0, 0)),
            pl.BlockSpec((1, C), lambda i: (0, 0)),
            pl.BlockSpec((N, Cc - C), lambda i: (0, 0)),
            pl.BlockSpec((Cc, KO), lambda i: (0, 0)),
            pl.BlockSpec((Cc, KO), lambda i: (0, 0)),
            pl.BlockSpec((Cc, C3), lambda i: (0, 0)),
            pl.BlockSpec((1, C3), lambda i: (0, 0)),
        ],
        out_specs=[
            pl.BlockSpec((BN, KO), lambda i: (i, 0)),
            pl.BlockSpec((BN, KO), lambda i: (i, 0)),
            pl.BlockSpec((N, C3), lambda i: (0, 0)),
        ],
        out_shape=[jax.ShapeDtypeStruct((N, KO), jnp.bfloat16),
                   jax.ShapeDtypeStruct((N, KO), jnp.bfloat16),
                   jax.ShapeDtypeStruct((N, C3), jnp.float32)],
        scratch_shapes=[pltpu.VMEM((N, Cc), jnp.float32)],
    )(parts, r, cinv, g, be, x, W3fa, W3fb, root3, b3)


def _make_sc_bag(N, N_acc, C_out, E_pad, G, add_cnt, n_t=1):
    C_msg = n_t * C_out + (32 if add_cnt else 0)
    CH = E_pad // (NW * G)
    PT = E_pad // NW
    PR = N_acc // NS
    CB = C_out // 16
    assert CH % 4 == 0 and PR % G == 0
    mesh = plsc.VectorSubcoreMesh(core_axis_name="c", subcore_axis_name="s",
                                  num_cores=NC, num_subcores=NS)

    @functools.partial(
        pl.kernel,
        out_type=jax.ShapeDtypeStruct((NC, N_acc, C_msg), jnp.bfloat16),
        mesh=mesh,
        compiler_params=pltpu.CompilerParams(use_tc_tiling_on_sc=False,
                                             needs_layout_passes=False),
        scratch_types=[
            pltpu.VMEM((8, G), jnp.float32),
            pltpu.VMEM((8, G), jnp.float32),
            pltpu.VMEM((8, G), jnp.int32),
            pltpu.VMEM((8, G), jnp.int32),
            pltpu.VMEM((G,), jnp.int32),
            pltpu.VMEM((G,), jnp.int32),
            pltpu.VMEM((G,), jnp.int32),
            pltpu.VMEM((G,), jnp.int32),
        ]
        + [pltpu.VMEM((8 * G, C_out), jnp.bfloat16)
           for _ in range(2 * n_t)]
        + [
            pltpu.VMEM((G, C_msg), jnp.bfloat16),
            pltpu.VMEM((G, C_msg), jnp.bfloat16),
            pltpu.VMEM_SHARED((N_acc, C_msg), jnp.bfloat16),
            pltpu.SemaphoreType.DMA,
            pltpu.SemaphoreType.DMA,
            pltpu.SemaphoreType.DMA,
            pltpu.SemaphoreType.DMA,
        ],
    )
    def sc_bag(*refs):
        y_hbm = refs[:n_t]
        b_hbm, rid_hbm, dst_hbm, out_hbm = refs[n_t:n_t + 4]
        sc = list(refs[n_t + 4:])
        b_v = sc[0:2]
        rid_v = sc[2:4]
        d_v = sc[4:8]
        rows_flat = sc[8:8 + 2 * n_t]
        rows_v = [rows_flat[:n_t], rows_flat[n_t:]]
        msg_v = sc[8 + 2 * n_t:10 + 2 * n_t]
        acc = sc[10 + 2 * n_t]
        sg = sc[11 + 2 * n_t:13 + 2 * n_t]
        ss = sc[13 + 2 * n_t:15 + 2 * n_t]

        cid = lax.axis_index("c")
        tid = lax.axis_index("s")
        wid = cid * NS + tid

        zero16 = jnp.zeros((16,), jnp.float32)
        zero32b = jnp.zeros((32,), jnp.bfloat16)

        def zrow(g, _):
            for cb in range(C_msg // 32):
                msg_v[0][g, pl.ds(cb * 32, 32)] = zero32b
            return 0
        lax.fori_loop(0, G, zrow, 0)

        for r0 in range(0, PR, G):
            pltpu.sync_copy(msg_v[0], acc.at[pl.ds(tid * PR + r0, G)])

        if add_cnt:
            one0f = jnp.where(lax.iota(jnp.int32, 16) == 0, 1.0, 0.0)
            one0 = plsc.pack(one0f, zero16,
                             format=plsc.PackFormat.INTERLEAVED)

            def crow(g, _):
                msg_v[0][g, pl.ds(n_t * C_out, 32)] = one0
                msg_v[1][g, pl.ds(n_t * C_out, 32)] = one0
                return 0
            lax.fori_loop(0, G, crow, 0)

        plsc.subcore_barrier()

        bidx = [jnp.full((16, 1), lane, jnp.int32) for lane in range(16)]
        gdn = lax.GatherDimensionNumbers(
            offset_dims=(), collapsed_slice_dims=(0,), start_index_map=(0,))

        def fetch(c, s2, s4):
            base = wid * PT + c * G
            pltpu.sync_copy(b_hbm.at[:, pl.ds(base, G)], b_v[s2])
            pltpu.sync_copy(rid_hbm.at[:, pl.ds(base, G)], rid_v[s2])
            pltpu.sync_copy(dst_hbm.at[pl.ds(base, G)], d_v[s4])
            for t in range(n_t):
                for s in range(8):
                    pltpu.async_copy(y_hbm[t].at[rid_v[s2].at[s]],
                                     rows_v[s2][t].at[pl.ds(s * G, G)], sg[s2])

        def wait_gathers(s2):
            for t in range(n_t):
                for s in range(8):
                    pltpu.make_async_copy(y_hbm[t].at[rid_v[s2].at[s]],
                                          rows_v[s2][t].at[pl.ds(s * G, G)],
                                          sg[s2]).wait()

        def wait_scatter(s2, s4):
            pltpu.make_async_copy(msg_v[s2], acc.at[d_v[s4]], ss[s2]).wait()

        def compute(s2):
            def grp(g16, _):
                gbase = g16 * 16
                bvs = [b_v[s2][s, pl.ds(gbase, 16)] for s in range(8)]
                for j in range(16):
                    g = gbase + j
                    accv = [zero16] * (n_t * CB)
                    for s in range(8):
                        bs = lax.gather(
                            bvs[s], bidx[j], gdn, slice_sizes=(1,),
                            mode=lax.GatherScatterMode.PROMISE_IN_BOUNDS)
                        row = s * G + g
                        for t in range(n_t):
                            for w in range(CB // 2):
                                pa, pb = plsc.unpack(
                                    rows_v[s2][t][row, pl.ds(w * 32, 32)],
                                    format=plsc.PackFormat.INTERLEAVED)
                                a0 = t * CB + 2 * w
                                accv[a0] = accv[a0] + bs * pa
                                accv[a0 + 1] = accv[a0 + 1] + bs * pb
                    for w in range(n_t * CB // 2):
                        msg_v[s2][g, pl.ds(w * 32, 32)] = plsc.pack(
                            accv[2 * w], accv[2 * w + 1],
                            format=plsc.PackFormat.INTERLEAVED)
                return 0
            lax.fori_loop(0, G // 16, grp, 0)

        fetch(0, 0, 0)

        def quad(c4, _):
            for ph in range(4):
                c = c4 * 4 + ph
                s2 = ph % 2
                @pl.when(c >= 2)
                def _():
                    wait_scatter(s2, (ph + 2) % 4)

                @pl.when(c + 1 < CH)
                def _():
                    fetch(c + 1, 1 - s2, (ph + 1) % 4)
                wait_gathers(s2)
                compute(s2)
                pltpu.async_copy(msg_v[s2], acc.at[d_v[ph]], ss[s2], add=True)
            return 0
        lax.fori_loop(0, CH // 4, quad, 0)

        wait_scatter(0, 2)
        wait_scatter(1, 3)

        plsc.subcore_barrier()
        pltpu.sync_copy(acc.at[pl.ds(tid * PR, PR)],
                        out_hbm.at[cid, pl.ds(tid * PR, PR)])

    return sc_bag


def _combine1(parts, r, g, be, C):
    N = r.shape[0]

    def body(p_ref, r_ref, g_ref, be_ref, h_ref, ci_ref):
        p0 = p_ref[0, :N]
        p1 = p_ref[1, :N]
        s = p0[:, :C] + p1[:, :C]
        cnt = p0[:, C:C + 1] + p1[:, C:C + 1]
        cinv = 1.0 / jnp.maximum(cnt, 1.0)
        out = s * cinv + r_ref[...]
        m = jnp.mean(out, axis=0, keepdims=True)
        var = jnp.mean(out * out, axis=0, keepdims=True) - m * m
        xn = (out - m) * lax.rsqrt(var + 1e-5) * g_ref[...] + be_ref[...]
        h_ref[...] = jnp.where(xn > 0, xn, jnp.exp(xn) - 1.0)
        ci_ref[...] = cinv

    return pl.pallas_call(
        body,
        out_shape=[jax.ShapeDtypeStruct((N, C), jnp.float32),
                   jax.ShapeDtypeStruct((N, 1), jnp.float32)],
    )(parts, r, g, be)


def _combine2(parts, r, cinv, g, be, C):
    N = r.shape[0]

    def body(p_ref, r_ref, ci_ref, g_ref, be_ref, h_ref):
        out = (p_ref[0, :N] + p_ref[1, :N]) * ci_ref[...] + r_ref[...]
        m = jnp.mean(out, axis=0, keepdims=True)
        var = jnp.mean(out * out, axis=0, keepdims=True) - m * m
        xn = (out - m) * lax.rsqrt(var + 1e-5) * g_ref[...] + be_ref[...]
        h_ref[...] = jnp.where(xn > 0, xn, jnp.exp(xn) - 1.0)

    return pl.pallas_call(
        body,
        out_shape=jax.ShapeDtypeStruct((N, C), jnp.float32),
    )(parts, r, cinv, g, be)


def _combine3(parts, r, cinv, C):
    N = r.shape[0]

    def body(p_ref, r_ref, ci_ref, o_ref):
        p = (p_ref[0, :N].astype(jnp.float32)
             + p_ref[1, :N].astype(jnp.float32))
        o_ref[...] = p * ci_ref[...] + r_ref[...]

    return pl.pallas_call(
        body,
        out_shape=jax.ShapeDtypeStruct((N, C), jnp.float32),
    )(parts, r, cinv)


def kernel(res, edge_index, edge_attr, x, W1, root1, b1, g1, be1,
           W2, root2, b2, g2, be2, W3, root3, b3):
    N = res.shape[0]
    E = edge_index.shape[1]
    d_in = res.shape[1]
    mid = root1.shape[1]
    d_out = root3.shape[1]

    E_pad = ((E + NW * 128 - 1) // (NW * 128)) * (NW * 128)
    N_acc = ((N + 1 + 1023) // 1024) * 1024

    src = edge_index[0].astype(jnp.int32)
    dst = edge_index[1].astype(jnp.int32)

    EC = E_pad // 8
    pad_e = E_pad - E
    attr_p = jnp.concatenate(
        [edge_attr.astype(jnp.float32),
         jnp.zeros((pad_e, DIM), jnp.float32)], axis=0)
    attr_r = attr_p.T.reshape(DIM, 8, EC)
    src_r = jnp.concatenate([src, jnp.zeros((pad_e,), jnp.int32)]).reshape(8, EC)
    dst_p = jnp.concatenate([dst, jnp.full((pad_e,), N, jnp.int32)])

    b_o, rid_o = _basis(attr_r, src_r, E)
    b8 = b_o.reshape(8, E_pad)
    rid8 = rid_o.reshape(8, E_pad)

    ho = d_out // 2
    W1f = W1.transpose(1, 0, 2).reshape(d_in, KD * mid)
    W2f = W2.transpose(1, 0, 2).reshape(mid, KD * 2 * mid)
    W3fa = W3[:, :, :ho].transpose(1, 0, 2).reshape(2 * mid + DIM, KD * ho)
    W3fb = W3[:, :, ho:].transpose(1, 0, 2).reshape(2 * mid + DIM, KD * ho)

    y1, r1 = _mm(res, [W1f], root1, b1[None], 400)
    bag1 = _make_sc_bag(N, N_acc, mid, E_pad, 128, True)
    parts1 = bag1(y1.reshape(N * KD, mid), b8, rid8, dst_p)

    y2, r2, cinv = _fuse1(parts1, r1, g1[None], be1[None],
                          W2f, root2, b2[None], mid, 400)
    bag2 = _make_sc_bag(N, N_acc, 2 * mid, E_pad, 128, False)
    parts2 = bag2(y2.reshape(N * KD, 2 * mid), b8, rid8, dst_p)

    y3a, y3b, r3 = _fuse2(parts2, r2, cinv, g2[None], be2[None],
                          x.astype(jnp.float32), W3fa, W3fb, root3, b3[None],
                          2 * mid, 400)
    bag3 = _make_sc_bag(N, N_acc, ho, E_pad, 64, False, n_t=2)
    parts3 = bag3(y3a.reshape(N * KD, ho), y3b.reshape(N * KD, ho),
                  b8, rid8, dst_p)
    return _combine3(parts3, r3, cinv, d_out)


# speedup vs baseline: 1.1497x; 1.1497x over previous
import functools

import jax
import jax.numpy as jnp
import numpy as np
from jax import lax
from jax.experimental import pallas as pl
from jax.experimental.pallas import tpu as pltpu
from jax.experimental.pallas import tpu_sc as plsc

K = 3
DIM = 3
KD = K ** DIM
NC = 2
NS = 16
NW = NC * NS


def _basis(attr_r, src_r, E):
    _, R, EC = attr_r.shape

    def body(a_ref, s_ref, b_ref, rid_ref):
        src = s_ref[...]
        fracs = []
        los = []
        for d in range(DIM):
            v = a_ref[d] * float(K)
            lo = jnp.floor(v)
            fracs.append(v - lo)
            los.append(lo.astype(jnp.int32))
        e_id = (lax.broadcasted_iota(jnp.int32, (R, EC), 0) * EC
                + lax.broadcasted_iota(jnp.int32, (R, EC), 1))
        valid = e_id < E
        for s in range(8):
            b = None
            idx = None
            for d in range(DIM):
                bit = (s >> d) & 1
                f = fracs[d] if bit else 1.0 - fracs[d]
                b = f if b is None else b * f
                t = lax.rem(los[d] + bit, K) * (K ** d)
                idx = t if idx is None else idx + t
            b = jnp.where(valid, b, 0.0)
            b_ref[s] = b
            rid_ref[s] = src * KD + idx

    return pl.pallas_call(
        body,
        out_shape=[jax.ShapeDtypeStruct((8, R, EC), jnp.float32),
                   jax.ShapeDtypeStruct((8, R, EC), jnp.int32)],
    )(attr_r, src_r)


def _mm(x, Wfs, root, bias, block_n):
    N, Cin = x.shape
    KOs = [Wf.shape[1] for Wf in Wfs]
    C = root.shape[1]
    nw = len(Wfs)

    def body(x_ref, *refs):
        w_refs = refs[:nw]
        rt_ref, b_ref = refs[nw], refs[nw + 1]
        y_refs = refs[nw + 2:nw + 2 + nw]
        r_ref = refs[-1]
        xb = x_ref[...]
        for w_ref, y_ref in zip(w_refs, y_refs):
            y_ref[...] = jnp.dot(
                xb, w_ref[...],
                preferred_element_type=jnp.float32).astype(jnp.bfloat16)
        r_ref[...] = (jnp.dot(xb, rt_ref[...], preferred_element_type=jnp.float32)
                      + b_ref[...])

    return pl.pallas_call(
        body,
        grid=(N // block_n,),
        in_specs=[pl.BlockSpec((block_n, Cin), lambda i: (i, 0))]
        + [pl.BlockSpec((Cin, KO), lambda i: (0, 0)) for KO in KOs]
        + [
            pl.BlockSpec((Cin, C), lambda i: (0, 0)),
            pl.BlockSpec((1, C), lambda i: (0, 0)),
        ],
        out_specs=[pl.BlockSpec((block_n, KO), lambda i: (i, 0)) for KO in KOs]
        + [pl.BlockSpec((block_n, C), lambda i: (i, 0))],
        out_shape=[jax.ShapeDtypeStruct((N, KO), jnp.bfloat16) for KO in KOs]
        + [jax.ShapeDtypeStruct((N, C), jnp.float32)],
    )(x, *Wfs, root, bias)


def _fuse1(parts, r, g, be, W2f, root2, b2, C, BN):
    N = r.shape[0]
    _, N_acc, C_msg = parts.shape
    KO = W2f.shape[1]
    C2 = root2.shape[1]

    def body(p_ref, r_ref, g_ref, be_ref, w_ref, rt_ref, b2_ref,
             y_ref, r2_ref, ci_ref, h_sc):
        i = pl.program_id(0)

        @pl.when(i == 0)
        def _():
            p0 = p_ref[0, :N].astype(jnp.float32)
            p1 = p_ref[1, :N].astype(jnp.float32)
            cnt = p0[:, C:C + 1] + p1[:, C:C + 1]
            cinv = 1.0 / jnp.maximum(cnt, 1.0)
            out = (p0[:, :C] + p1[:, :C]) * cinv + r_ref[...]
            m = jnp.mean(out, axis=0, keepdims=True)
            var = jnp.mean(out * out, axis=0, keepdims=True) - m * m
            xn = (out - m) * lax.rsqrt(var + 1e-5) * g_ref[...] + be_ref[...]
            h = jnp.where(xn > 0, xn, jnp.exp(xn) - 1.0)
            h_sc[...] = h
            ci_ref[...] = cinv
            r2_ref[...] = (jnp.dot(h, rt_ref[...],
                                   preferred_element_type=jnp.float32)
                           + b2_ref[...])
        y_ref[...] = jnp.dot(
            h_sc[pl.ds(i * BN, BN), :], w_ref[...],
            preferred_element_type=jnp.float32).astype(jnp.bfloat16)

    return pl.pallas_call(
        body,
        grid=(N // BN,),
        in_specs=[
            pl.BlockSpec((2, N_acc, C_msg), lambda i: (0, 0, 0)),
            pl.BlockSpec((N, C), lambda i: (0, 0)),
            pl.BlockSpec((1, C), lambda i: (0, 0)),
            pl.BlockSpec((1, C), lambda i: (0, 0)),
            pl.BlockSpec((C, KO), lambda i: (0, 0)),
            pl.BlockSpec((C, C2), lambda i: (0, 0)),
            pl.BlockSpec((1, C2), lambda i: (0, 0)),
        ],
        out_specs=[
            pl.BlockSpec((BN, KO), lambda i: (i, 0)),
            pl.BlockSpec((N, C2), lambda i: (0, 0)),
            pl.BlockSpec((N, 1), lambda i: (0, 0)),
        ],
        out_shape=[jax.ShapeDtypeStruct((N, KO), jnp.bfloat16),
                   jax.ShapeDtypeStruct((N, C2), jnp.float32),
                   jax.ShapeDtypeStruct((N, 1), jnp.float32)],
        scratch_shapes=[pltpu.VMEM((N, C), jnp.float32)],
    )(parts, r, g, be, W2f, root2, b2)


def _fuse2(parts, r, cinv, g, be, x, W3fa, W3fb, root3, b3, C, BN):
    N = r.shape[0]
    _, N_acc, C_msg = parts.shape
    KO = W3fa.shape[1]
    C3 = root3.shape[1]
    Cc = C + x.shape[1]

    def body(p_ref, r_ref, ci_ref, g_ref, be_ref, x_ref, wa_ref, wb_ref,
             rt_ref, b3_ref, ya_ref, yb_ref, r3_ref, h_sc):
        i = pl.program_id(0)

        @pl.when(i == 0)
        def _():
            out = ((p_ref[0, :N].astype(jnp.float32)
                    + p_ref[1, :N].astype(jnp.float32)) * ci_ref[...]
                   + r_ref[...])
            m = jnp.mean(out, axis=0, keepdims=True)
            var = jnp.mean(out * out, axis=0, keepdims=True) - m * m
            xn = (out - m) * lax.rsqrt(var + 1e-5) * g_ref[...] + be_ref[...]
            h2 = jnp.where(xn > 0, xn, jnp.exp(xn) - 1.0)
            hc = jnp.concatenate([h2, x_ref[...]], axis=1)
            h_sc[...] = hc
            r3_ref[...] = (jnp.dot(hc, rt_ref[...],
                                   preferred_element_type=jnp.float32)
                           + b3_ref[...])
        hc = h_sc[pl.ds(i * BN, BN), :]
        ya_ref[...] = jnp.dot(
            hc, wa_ref[...],
            preferred_element_type=jnp.float32).astype(jnp.bfloat16)
        yb_ref[...] = jnp.dot(
            hc, wb_ref[...],
            preferred_element_type=jnp.float32).astype(jnp.bfloat16)

    return pl.pallas_call(
        body,
        grid=(N // BN,),
        in_specs=[
            pl.BlockSpec((2, N_acc, C_msg), lambda i: (0, 0, 0)),
            pl.BlockSpec((N, C), lambda i: (0, 0)),
            pl.BlockSpec((N, 1), lambda i: (0, 0)),
            pl.BlockSpec((1, C), lambda i: (0, 0)),
            pl.BlockSpec((1, C), lambda i: (0, 0)),
            pl.BlockSpec((N, Cc - C), lambda i: (0, 0)),
            pl.BlockSpec((Cc, KO), lambda i: (0, 0)),
            pl.BlockSpec((Cc, KO), lambda i: (0, 0)),
            pl.BlockSpec((Cc, C3), lambda i: (0, 0)),
            pl.BlockSpec((1, C3), lambda i: (0, 0)),
        ],
        out_specs=[
            pl.BlockSpec((BN, KO), lambda i: (i, 0)),
            pl.BlockSpec((BN, KO), lambda i: (i, 0)),
            pl.BlockSpec((N, C3), lambda i: (0, 0)),
        ],
        out_shape=[jax.ShapeDtypeStruct((N, KO), jnp.bfloat16),
                   jax.ShapeDtypeStruct((N, KO), jnp.bfloat16),
                   jax.ShapeDtypeStruct((N, C3), jnp.float32)],
        scratch_shapes=[pltpu.VMEM((N, Cc), jnp.float32)],
    )(parts, r, cinv, g, be, x, W3fa, W3fb, root3, b3)


def _make_sc_bag(N, N_acc, C_out, E_pad, G, add_cnt, n_t=1):
    C_msg = n_t * C_out + (32 if add_cnt else 0)
    CH = E_pad // (NW * G)
    PT = E_pad // NW
    PR = N_acc // NS
    CB = C_out // 16
    assert CH % 4 == 0 and PR % G == 0
    mesh = plsc.VectorSubcoreMesh(core_axis_name="c", subcore_axis_name="s",
                                  num_cores=NC, num_subcores=NS)

    @functools.partial(
        pl.kernel,
        out_type=jax.ShapeDtypeStruct((NC, N_acc, C_msg), jnp.bfloat16),
        mesh=mesh,
        compiler_params=pltpu.CompilerParams(use_tc_tiling_on_sc=False,
                                             needs_layout_passes=False),
        scratch_types=[
            pltpu.VMEM((8, G), jnp.float32),
            pltpu.VMEM((8, G), jnp.float32),
            pltpu.VMEM((8, G), jnp.int32),
            pltpu.VMEM((8, G), jnp.int32),
            pltpu.VMEM((G,), jnp.int32),
            pltpu.VMEM((G,), jnp.int32),
            pltpu.VMEM((G,), jnp.int32),
            pltpu.VMEM((G,), jnp.int32),
        ]
        + [pltpu.VMEM((8 * G, C_out), jnp.bfloat16)
           for _ in range(2 * n_t)]
        + [
            pltpu.VMEM((G, C_msg), jnp.bfloat16),
            pltpu.VMEM((G, C_msg), jnp.bfloat16),
            pltpu.VMEM_SHARED((N_acc, C_msg), jnp.bfloat16),
            pltpu.SemaphoreType.DMA,
            pltpu.SemaphoreType.DMA,
            pltpu.SemaphoreType.DMA,
            pltpu.SemaphoreType.DMA,
        ],
    )
    def sc_bag(*refs):
        y_hbm = refs[:n_t]
        b_hbm, rid_hbm, dst_hbm, out_hbm = refs[n_t:n_t + 4]
        sc = list(refs[n_t + 4:])
        b_v = sc[0:2]
        rid_v = sc[2:4]
        d_v = sc[4:8]
        rows_flat = sc[8:8 + 2 * n_t]
        rows_v = [rows_flat[:n_t], rows_flat[n_t:]]
        msg_v = sc[8 + 2 * n_t:10 + 2 * n_t]
        acc = sc[10 + 2 * n_t]
        sg = sc[11 + 2 * n_t:13 + 2 * n_t]
        ss = sc[13 + 2 * n_t:15 + 2 * n_t]

        cid = lax.axis_index("c")
        tid = lax.axis_index("s")
        wid = cid * NS + tid

        zero16 = jnp.zeros((16,), jnp.float32)
        zero32b = jnp.zeros((32,), jnp.bfloat16)

        def zrow(g, _):
            for cb in range(C_msg // 32):
                msg_v[0][g, pl.ds(cb * 32, 32)] = zero32b
            return 0
        lax.fori_loop(0, G, zrow, 0)

        for r0 in range(0, PR, G):
            pltpu.sync_copy(msg_v[0], acc.at[pl.ds(tid * PR + r0, G)])

        if add_cnt:
            one0f = jnp.where(lax.iota(jnp.int32, 16) == 0, 1.0, 0.0)
            one0 = plsc.pack(one0f, zero16,
                             format=plsc.PackFormat.INTERLEAVED)

            def crow(g, _):
                msg_v[0][g, pl.ds(n_t * C_out, 32)] = one0
                msg_v[1][g, pl.ds(n_t * C_out, 32)] = one0
                return 0
            lax.fori_loop(0, G, crow, 0)

        plsc.subcore_barrier()

        bidx = [jnp.full((16, 1), lane, jnp.int32) for lane in range(16)]
        gdn = lax.GatherDimensionNumbers(
            offset_dims=(), collapsed_slice_dims=(0,), start_index_map=(0,))

        def fetch(c, s2, s4):
            base = wid * PT + c * G
            pltpu.sync_copy(b_hbm.at[:, pl.ds(base, G)], b_v[s2])
            pltpu.sync_copy(rid_hbm.at[:, pl.ds(base, G)], rid_v[s2])
            pltpu.sync_copy(dst_hbm.at[pl.ds(base, G)], d_v[s4])
            for t in range(n_t):
                for s in range(8):
                    pltpu.async_copy(y_hbm[t].at[rid_v[s2].at[s]],
                                     rows_v[s2][t].at[pl.ds(s * G, G)], sg[s2])

        def wait_gathers(s2):
            for t in range(n_t):
                for s in range(8):
                    pltpu.make_async_copy(y_hbm[t].at[rid_v[s2].at[s]],
                                          rows_v[s2][t].at[pl.ds(s * G, G)],
                                          sg[s2]).wait()

        def wait_scatter(s2, s4):
            pltpu.make_async_copy(msg_v[s2], acc.at[d_v[s4]], ss[s2]).wait()

        def compute(s2):
            def grp(g16, _):
                gbase = g16 * 16
                bvs = [b_v[s2][s, pl.ds(gbase, 16)] for s in range(8)]
                for j in range(16):
                    g = gbase + j
                    accv = [zero16] * (n_t * CB)
                    for s in range(8):
                        bs = lax.gather(
                            bvs[s], bidx[j], gdn, slice_sizes=(1,),
                            mode=lax.GatherScatterMode.PROMISE_IN_BOUNDS)
                        row = s * G + g
                        for t in range(n_t):
                            for w in range(CB // 2):
                                pa, pb = plsc.unpack(
                                    rows_v[s2][t][row, pl.ds(w * 32, 32)],
                                    format=plsc.PackFormat.INTERLEAVED)
                                a0 = t * CB + 2 * w
                                accv[a0] = accv[a0] + bs * pa
                                accv[a0 + 1] = accv[a0 + 1] + bs * pb
                    for w in range(n_t * CB // 2):
                        msg_v[s2][g, pl.ds(w * 32, 32)] = plsc.pack(
                            accv[2 * w], accv[2 * w + 1],
                            format=plsc.PackFormat.INTERLEAVED)
                return 0
            lax.fori_loop(0, G // 16, grp, 0)

        fetch(0, 0, 0)

        def quad(c4, _):
            for ph in range(4):
                c = c4 * 4 + ph
                s2 = ph % 2
                @pl.when(c >= 2)
                def _():
                    wait_scatter(s2, (ph + 2) % 4)

                @pl.when(c + 1 < CH)
                def _():
                    fetch(c + 1, 1 - s2, (ph + 1) % 4)
                wait_gathers(s2)
                compute(s2)
                pltpu.async_copy(msg_v[s2], acc.at[d_v[ph]], ss[s2], add=True)
            return 0
        lax.fori_loop(0, CH // 4, quad, 0)

        wait_scatter(0, 2)
        wait_scatter(1, 3)

        plsc.subcore_barrier()
        pltpu.sync_copy(acc.at[pl.ds(tid * PR, PR)],
                        out_hbm.at[cid, pl.ds(tid * PR, PR)])

    return sc_bag


def _combine1(parts, r, g, be, C):
    N = r.shape[0]

    def body(p_ref, r_ref, g_ref, be_ref, h_ref, ci_ref):
        p0 = p_ref[0, :N]
        p1 = p_ref[1, :N]
        s = p0[:, :C] + p1[:, :C]
        cnt = p0[:, C:C + 1] + p1[:, C:C + 1]
        cinv = 1.0 / jnp.maximum(cnt, 1.0)
        out = s * cinv + r_ref[...]
        m = jnp.mean(out, axis=0, keepdims=True)
        var = jnp.mean(out * out, axis=0, keepdims=True) - m * m
        xn = (out - m) * lax.rsqrt(var + 1e-5) * g_ref[...] + be_ref[...]
        h_ref[...] = jnp.where(xn > 0, xn, jnp.exp(xn) - 1.0)
        ci_ref[...] = cinv

    return pl.pallas_call(
        body,
        out_shape=[jax.ShapeDtypeStruct((N, C), jnp.float32),
                   jax.ShapeDtypeStruct((N, 1), jnp.float32)],
    )(parts, r, g, be)


def _combine2(parts, r, cinv, g, be, C):
    N = r.shape[0]

    def body(p_ref, r_ref, ci_ref, g_ref, be_ref, h_ref):
        out = (p_ref[0, :N] + p_ref[1, :N]) * ci_ref[...] + r_ref[...]
        m = jnp.mean(out, axis=0, keepdims=True)
        var = jnp.mean(out * out, axis=0, keepdims=True) - m * m
        xn = (out - m) * lax.rsqrt(var + 1e-5) * g_ref[...] + be_ref[...]
        h_ref[...] = jnp.where(xn > 0, xn, jnp.exp(xn) - 1.0)

    return pl.pallas_call(
        body,
        out_shape=jax.ShapeDtypeStruct((N, C), jnp.float32),
    )(parts, r, cinv, g, be)


def _combine3(parts, r, cinv, C):
    N = r.shape[0]

    def body(pa_ref, pb_ref, r_ref, ci_ref, o_ref):
        ci = ci_ref[...]
        rr = r_ref[...]
        ha = ((pa_ref[0, :N].astype(jnp.float32)
               + pa_ref[1, :N].astype(jnp.float32)) * ci + rr[:, :C // 2])
        hb = ((pb_ref[0, :N].astype(jnp.float32)
               + pb_ref[1, :N].astype(jnp.float32)) * ci + rr[:, C // 2:])
        o_ref[...] = jnp.concatenate([ha, hb], axis=1)

    return pl.pallas_call(
        body,
        out_shape=jax.ShapeDtypeStruct((N, C), jnp.float32),
    )(*parts, r, cinv)


def kernel(res, edge_index, edge_attr, x, W1, root1, b1, g1, be1,
           W2, root2, b2, g2, be2, W3, root3, b3):
    N = res.shape[0]
    E = edge_index.shape[1]
    d_in = res.shape[1]
    mid = root1.shape[1]
    d_out = root3.shape[1]

    E_pad = ((E + NW * 128 - 1) // (NW * 128)) * (NW * 128)
    N_acc = ((N + 1 + 1023) // 1024) * 1024

    src = edge_index[0].astype(jnp.int32)
    dst = edge_index[1].astype(jnp.int32)

    EC = E_pad // 8
    pad_e = E_pad - E
    attr_p = jnp.concatenate(
        [edge_attr.astype(jnp.float32),
         jnp.zeros((pad_e, DIM), jnp.float32)], axis=0)
    attr_r = attr_p.T.reshape(DIM, 8, EC)
    src_r = jnp.concatenate([src, jnp.zeros((pad_e,), jnp.int32)]).reshape(8, EC)
    dst_p = jnp.concatenate([dst, jnp.full((pad_e,), N, jnp.int32)])

    b_o, rid_o = _basis(attr_r, src_r, E)
    b8 = b_o.reshape(8, E_pad)
    rid8 = rid_o.reshape(8, E_pad)

    ho = d_out // 2
    W1f = W1.transpose(1, 0, 2).reshape(d_in, KD * mid)
    W2f = W2.transpose(1, 0, 2).reshape(mid, KD * 2 * mid)
    W3fa = W3[:, :, :ho].transpose(1, 0, 2).reshape(2 * mid + DIM, KD * ho)
    W3fb = W3[:, :, ho:].transpose(1, 0, 2).reshape(2 * mid + DIM, KD * ho)

    y1, r1 = _mm(res, [W1f], root1, b1[None], 400)
    bag1 = _make_sc_bag(N, N_acc, mid, E_pad, 128, True)
    parts1 = bag1(y1.reshape(N * KD, mid), b8, rid8, dst_p)

    y2, r2, cinv = _fuse1(parts1, r1, g1[None], be1[None],
                          W2f, root2, b2[None], mid, 400)
    bag2 = _make_sc_bag(N, N_acc, 2 * mid, E_pad, 128, False)
    parts2 = bag2(y2.reshape(N * KD, 2 * mid), b8, rid8, dst_p)

    y3a, y3b, r3 = _fuse2(parts2, r2, cinv, g2[None], be2[None],
                          x.astype(jnp.float32), W3fa, W3fb, root3, b3[None],
                          2 * mid, 400)
    bag3 = _make_sc_bag(N, N_acc, ho, E_pad, 128, False)
    parts3a = bag3(y3a.reshape(N * KD, ho), b8, rid8, dst_p)
    parts3b = bag3(y3b.reshape(N * KD, ho), b8, rid8, dst_p)
    return _combine3([parts3a, parts3b], r3, cinv, d_out)
